# baseline (device time: 114189 ns/iter reference)
import jax
import jax.numpy as jnp
from jax import lax
from jax.experimental import pallas as pl
from jax.experimental.pallas import tpu as pltpu

N_DEV = 4
W_CHUNKS = 8
N_HOPS = N_DEV - 1


def kernel(x, w_mat, scale_x, scale_w):
    m_per, k = x.shape
    _, n_total = w_mat.shape
    n_per = n_total // N_DEV
    m_total = m_per * N_DEV
    half = m_per // 2
    qrt = m_per // 4
    kc = k // W_CHUNKS

    def body(x_ref, w_ref, sx_ref, sw_ref, out_ref,
             xg, w8, wstage, xstage, stage,
             ws_sems, xin_sems, rs_sems, rr_sems, ls_sems, lr_sems,
             copy_sems):
        my = lax.axis_index("i")
        left = lax.rem(my + (N_DEV - 1), N_DEV)
        right = lax.rem(my + 1, N_DEV)

        barrier = pltpu.get_barrier_semaphore()
        for nbr in (left, right):
            pl.semaphore_signal(barrier, inc=1, device_id=(nbr,),
                                device_id_type=pl.DeviceIdType.MESH)
        pl.semaphore_wait(barrier, 2)

        scale = sx_ref[0] * sw_ref[0]

        xcps = []
        for q in range(4):
            cp = pltpu.make_async_copy(
                x_ref.at[pl.ds(q * qrt, qrt), :],
                xstage.at[q], xin_sems.at[q])
            cp.start()
            xcps.append(cp)

        def x_quarter_in(q):
            xcps[q].wait()
            xg[pl.ds(my * m_per + q * qrt, qrt), :] = (
                xstage[q].astype(jnp.float8_e4m3fn))

        wcps = [None] * W_CHUNKS

        def start_wchunk(c):
            cp = pltpu.make_async_copy(
                w_ref.at[pl.ds(c * kc, kc), pl.ds(my * n_per, n_per)],
                wstage.at[c % 2], ws_sems.at[c % 2])
            cp.start()
            wcps[c] = cp

        start_wchunk(0)
        start_wchunk(1)

        def r_sub(h, j):
            o = lax.rem(my + (N_DEV - h), N_DEV)
            rows = pl.ds(o * m_per + j * qrt, qrt)
            return pltpu.make_async_remote_copy(
                src_ref=xg.at[rows, :], dst_ref=xg.at[rows, :],
                send_sem=rs_sems.at[2 * h + j], recv_sem=rr_sems.at[2 * h + j],
                device_id=(right,), device_id_type=pl.DeviceIdType.MESH)

        def l_sub(h, j):
            o = lax.rem(my + h, N_DEV)
            rows = pl.ds(o * m_per + half + j * qrt, qrt)
            return pltpu.make_async_remote_copy(
                src_ref=xg.at[rows, :], dst_ref=xg.at[rows, :],
                send_sem=ls_sems.at[2 * h + j], recv_sem=lr_sems.at[2 * h + j],
                device_id=(left,), device_id_type=pl.DeviceIdType.MESH)

        rsubs = {}
        lsubs = {}

        x_quarter_in(0)
        rsubs[(0, 0)] = r_sub(0, 0)
        rsubs[(0, 0)].start()
        x_quarter_in(2)
        lsubs[(0, 0)] = l_sub(0, 0)
        lsubs[(0, 0)].start()
        x_quarter_in(1)
        rsubs[(0, 1)] = r_sub(0, 1)
        rsubs[(0, 1)].start()
        x_quarter_in(3)
        lsubs[(0, 1)] = l_sub(0, 1)
        lsubs[(0, 1)].start()

        def w_chunk_in(c):
            wcps[c].wait()
            w8[pl.ds(c * kc, kc), :] = wstage[c % 2].astype(jnp.float8_e4m3fn)
            if c + 2 < W_CHUNKS:
                start_wchunk(c + 2)

        pending = [None, None]
        slot = [0]

        def compute_qrt(o, q):
            s = slot[0] & 1
            slot[0] += 1
            if pending[s] is not None:
                pending[s].wait()
            row = o * m_per + q * qrt
            acc = lax.dot_general(
                xg[pl.ds(row, qrt), :], w8[...],
                dimension_numbers=(((1,), (0,)), ((), ())),
                preferred_element_type=jnp.float32,
            )
            stage[s] = jnp.maximum(acc * scale, 0.0)
            cp = pltpu.make_async_copy(
                stage.at[s], out_ref.at[pl.ds(row, qrt), :],
                copy_sems.at[s])
            cp.start()
            pending[s] = cp

        def fwd_r(h, j):
            rsubs[(h - 1, j)].wait_recv()
            rsubs[(h, j)] = r_sub(h, j)
            rsubs[(h, j)].start()

        def fwd_l(h, j):
            lsubs[(h - 1, j)].wait_recv()
            lsubs[(h, j)] = l_sub(h, j)
            lsubs[(h, j)].start()

        o_am1 = lax.rem(my + (N_DEV - 1), N_DEV)
        o_ap1 = lax.rem(my + 1, N_DEV)
        o_2 = lax.rem(my + 2, N_DEV)

        w_chunk_in(0)
        w_chunk_in(1)
        fwd_r(1, 0)
        fwd_l(1, 0)
        w_chunk_in(2)
        w_chunk_in(3)
        fwd_r(1, 1)
        fwd_l(1, 1)
        w_chunk_in(4)
        w_chunk_in(5)
        fwd_r(2, 0)
        fwd_l(2, 0)
        w_chunk_in(6)
        w_chunk_in(7)
        compute_qrt(my, 0)
        compute_qrt(my, 1)
        fwd_r(2, 1)
        fwd_l(2, 1)
        compute_qrt(my, 2)
        compute_qrt(my, 3)
        compute_qrt(o_am1, 0)
        compute_qrt(o_am1, 1)
        compute_qrt(o_ap1, 2)
        compute_qrt(o_ap1, 3)
        compute_qrt(o_2, 0)
        compute_qrt(o_2, 2)
        compute_qrt(o_2, 1)
        compute_qrt(o_2, 3)

        rsubs[(2, 0)].wait_recv()
        compute_qrt(o_ap1, 0)
        lsubs[(2, 0)].wait_recv()
        compute_qrt(o_am1, 2)
        rsubs[(2, 1)].wait_recv()
        compute_qrt(o_ap1, 1)
        lsubs[(2, 1)].wait_recv()
        compute_qrt(o_am1, 3)

        for h in range(N_HOPS):
            for j in range(2):
                rsubs[(h, j)].wait_send()
                lsubs[(h, j)].wait_send()

        for p in pending:
            if p is not None:
                p.wait()

    return pl.pallas_call(
        body,
        out_shape=jax.ShapeDtypeStruct((m_total, n_per), jnp.float32),
        in_specs=[
            pl.BlockSpec(memory_space=pl.ANY),
            pl.BlockSpec(memory_space=pl.ANY),
            pl.BlockSpec(memory_space=pltpu.SMEM),
            pl.BlockSpec(memory_space=pltpu.SMEM),
        ],
        out_specs=pl.BlockSpec(memory_space=pl.ANY),
        scratch_shapes=[
            pltpu.VMEM((m_total, k), jnp.float8_e4m3fn),
            pltpu.VMEM((k, n_per), jnp.float8_e4m3fn),
            pltpu.VMEM((2, kc, n_per), jnp.float32),
            pltpu.VMEM((4, qrt, k), jnp.float32),
            pltpu.VMEM((2, qrt, n_per), jnp.float32),
            pltpu.SemaphoreType.DMA((2,)),
            pltpu.SemaphoreType.DMA((4,)),
            pltpu.SemaphoreType.DMA((2 * N_HOPS,)),
            pltpu.SemaphoreType.DMA((2 * N_HOPS,)),
            pltpu.SemaphoreType.DMA((2 * N_HOPS,)),
            pltpu.SemaphoreType.DMA((2 * N_HOPS,)),
            pltpu.SemaphoreType.DMA((2,)),
        ],
        compiler_params=pltpu.CompilerParams(
            collective_id=0, vmem_limit_bytes=100 * 1024 * 1024),
    )(x, w_mat, scale_x, scale_w)


# device time: 104093 ns/iter; 1.0970x vs baseline; 1.0970x over previous
import jax
import jax.numpy as jnp
from jax import lax
from jax.experimental import pallas as pl
from jax.experimental.pallas import tpu as pltpu

N_DEV = 4
W_CHUNKS = 8
N_HOPS = N_DEV - 1


def kernel(x, w_mat, scale_x, scale_w):
    m_per, k = x.shape
    _, n_total = w_mat.shape
    n_per = n_total // N_DEV
    m_total = m_per * N_DEV
    half = m_per // 2
    qrt = m_per // 4
    kc = k // W_CHUNKS

    def body(x_ref, w_ref, sx_ref, sw_ref, out_ref,
             xg, w8, wstage, xstage, stage,
             ws_sems, xin_sems, rs_sems, rr_sems, ls_sems, lr_sems,
             copy_sems):
        my = lax.axis_index("i")
        left = lax.rem(my + (N_DEV - 1), N_DEV)
        right = lax.rem(my + 1, N_DEV)

        barrier = pltpu.get_barrier_semaphore()
        for nbr in (left, right):
            pl.semaphore_signal(barrier, inc=1, device_id=(nbr,),
                                device_id_type=pl.DeviceIdType.MESH)
        pl.semaphore_wait(barrier, 2)

        scale = sx_ref[0] * sw_ref[0]

        xcps = []
        for q in range(4):
            cp = pltpu.make_async_copy(
                x_ref.at[pl.ds(q * qrt, qrt), :],
                xstage.at[q], xin_sems.at[q])
            cp.start()
            xcps.append(cp)

        def x_quarter_in(q):
            xcps[q].wait()
            xg[pl.ds(my * m_per + q * qrt, qrt), :] = (
                xstage[q].astype(jnp.float8_e4m3fn))

        wcps = [None] * W_CHUNKS

        def start_wchunk(c):
            cp = pltpu.make_async_copy(
                w_ref.at[pl.ds(c * kc, kc), pl.ds(my * n_per, n_per)],
                wstage.at[c % 2], ws_sems.at[c % 2])
            cp.start()
            wcps[c] = cp

        start_wchunk(0)
        start_wchunk(1)

        def r_sub(h, j):
            o = lax.rem(my + (N_DEV - h), N_DEV)
            rows = pl.ds(o * m_per + j * qrt, qrt)
            return pltpu.make_async_remote_copy(
                src_ref=xg.at[rows, :], dst_ref=xg.at[rows, :],
                send_sem=rs_sems.at[2 * h + j], recv_sem=rr_sems.at[2 * h + j],
                device_id=(right,), device_id_type=pl.DeviceIdType.MESH)

        def l_sub(h, j):
            o = lax.rem(my + h, N_DEV)
            rows = pl.ds(o * m_per + half + j * qrt, qrt)
            return pltpu.make_async_remote_copy(
                src_ref=xg.at[rows, :], dst_ref=xg.at[rows, :],
                send_sem=ls_sems.at[2 * h + j], recv_sem=lr_sems.at[2 * h + j],
                device_id=(left,), device_id_type=pl.DeviceIdType.MESH)

        rsubs = {}
        lsubs = {}

        x_quarter_in(0)
        rsubs[(0, 0)] = r_sub(0, 0)
        rsubs[(0, 0)].start()
        x_quarter_in(2)
        lsubs[(0, 0)] = l_sub(0, 0)
        lsubs[(0, 0)].start()
        x_quarter_in(1)
        rsubs[(0, 1)] = r_sub(0, 1)
        rsubs[(0, 1)].start()
        x_quarter_in(3)
        lsubs[(0, 1)] = l_sub(0, 1)
        lsubs[(0, 1)].start()

        def w_chunk_in(c):
            wcps[c].wait()
            w8[pl.ds(c * kc, kc), :] = wstage[c % 2].astype(jnp.float8_e4m3fn)
            if c + 2 < W_CHUNKS:
                start_wchunk(c + 2)

        pending = [None, None]
        slot = [0]

        def compute_qrt(o, q):
            s = slot[0] & 1
            slot[0] += 1
            if pending[s] is not None:
                pending[s].wait()
            row = o * m_per + q * qrt
            acc = lax.dot_general(
                xg[pl.ds(row, qrt), :], w8[...],
                dimension_numbers=(((1,), (0,)), ((), ())),
                preferred_element_type=jnp.float32,
            )
            stage[s] = jnp.maximum(acc * scale, 0.0)
            cp = pltpu.make_async_copy(
                stage.at[s], out_ref.at[pl.ds(row, qrt), :],
                copy_sems.at[s])
            cp.start()
            pending[s] = cp

        def fwd_r(h, j):
            rsubs[(h - 1, j)].wait_recv()
            rsubs[(h, j)] = r_sub(h, j)
            rsubs[(h, j)].start()

        def fwd_l(h, j):
            lsubs[(h - 1, j)].wait_recv()
            lsubs[(h, j)] = l_sub(h, j)
            lsubs[(h, j)].start()

        o_am1 = lax.rem(my + (N_DEV - 1), N_DEV)
        o_ap1 = lax.rem(my + 1, N_DEV)
        o_2 = lax.rem(my + 2, N_DEV)

        for c in range(W_CHUNKS):
            w_chunk_in(c)
        fwd_r(1, 0)
        fwd_l(1, 0)
        compute_qrt(my, 0)
        compute_qrt(my, 1)
        fwd_r(1, 1)
        fwd_l(1, 1)
        compute_qrt(my, 2)
        compute_qrt(my, 3)
        compute_qrt(o_am1, 0)
        fwd_r(2, 0)
        fwd_l(2, 0)
        compute_qrt(o_am1, 1)
        compute_qrt(o_ap1, 2)
        fwd_r(2, 1)
        fwd_l(2, 1)
        compute_qrt(o_ap1, 3)
        compute_qrt(o_2, 0)
        compute_qrt(o_2, 2)
        compute_qrt(o_2, 1)
        compute_qrt(o_2, 3)

        rsubs[(2, 0)].wait_recv()
        compute_qrt(o_ap1, 0)
        lsubs[(2, 0)].wait_recv()
        compute_qrt(o_am1, 2)
        rsubs[(2, 1)].wait_recv()
        compute_qrt(o_ap1, 1)
        lsubs[(2, 1)].wait_recv()
        compute_qrt(o_am1, 3)

        for h in range(N_HOPS):
            for j in range(2):
                rsubs[(h, j)].wait_send()
                lsubs[(h, j)].wait_send()

        for p in pending:
            if p is not None:
                p.wait()

    return pl.pallas_call(
        body,
        out_shape=jax.ShapeDtypeStruct((m_total, n_per), jnp.float32),
        in_specs=[
            pl.BlockSpec(memory_space=pl.ANY),
            pl.BlockSpec(memory_space=pl.ANY),
            pl.BlockSpec(memory_space=pltpu.SMEM),
            pl.BlockSpec(memory_space=pltpu.SMEM),
        ],
        out_specs=pl.BlockSpec(memory_space=pl.ANY),
        scratch_shapes=[
            pltpu.VMEM((m_total, k), jnp.float8_e4m3fn),
            pltpu.VMEM((k, n_per), jnp.float8_e4m3fn),
            pltpu.VMEM((2, kc, n_per), jnp.float32),
            pltpu.VMEM((4, qrt, k), jnp.float32),
            pltpu.VMEM((2, qrt, n_per), jnp.float32),
            pltpu.SemaphoreType.DMA((2,)),
            pltpu.SemaphoreType.DMA((4,)),
            pltpu.SemaphoreType.DMA((2 * N_HOPS,)),
            pltpu.SemaphoreType.DMA((2 * N_HOPS,)),
            pltpu.SemaphoreType.DMA((2 * N_HOPS,)),
            pltpu.SemaphoreType.DMA((2 * N_HOPS,)),
            pltpu.SemaphoreType.DMA((2,)),
        ],
        compiler_params=pltpu.CompilerParams(
            collective_id=0, vmem_limit_bytes=100 * 1024 * 1024),
    )(x, w_mat, scale_x, scale_w)


# device time: 104086 ns/iter; 1.0971x vs baseline; 1.0001x over previous
import jax
import jax.numpy as jnp
from jax import lax
from jax.experimental import pallas as pl
from jax.experimental.pallas import tpu as pltpu

N_DEV = 4
W_CHUNKS = 8
N_HOPS = N_DEV - 1


def kernel(x, w_mat, scale_x, scale_w):
    m_per, k = x.shape
    _, n_total = w_mat.shape
    n_per = n_total // N_DEV
    m_total = m_per * N_DEV
    half = m_per // 2
    qrt = m_per // 4
    kc = k // W_CHUNKS

    def body(x_ref, w_ref, sx_ref, sw_ref, out_ref,
             xg, w8, wstage, xstage, stage,
             ws_sems, xin_sems, rs_sems, rr_sems, ls_sems, lr_sems,
             copy_sems):
        my = lax.axis_index("i")
        left = lax.rem(my + (N_DEV - 1), N_DEV)
        right = lax.rem(my + 1, N_DEV)

        barrier = pltpu.get_barrier_semaphore()
        for nbr in (left, right):
            pl.semaphore_signal(barrier, inc=1, device_id=(nbr,),
                                device_id_type=pl.DeviceIdType.MESH)
        pl.semaphore_wait(barrier, 2)

        scale = sx_ref[0] * sw_ref[0]

        xcps = []
        for q in range(4):
            cp = pltpu.make_async_copy(
                x_ref.at[pl.ds(q * qrt, qrt), :],
                xstage.at[q], xin_sems.at[q])
            cp.start()
            xcps.append(cp)

        def x_quarter_in(q):
            xcps[q].wait()
            xg[pl.ds(my * m_per + q * qrt, qrt), :] = (
                xstage[q].astype(jnp.float8_e4m3fn))

        wcps = [None] * W_CHUNKS

        def start_wchunk(c):
            cp = pltpu.make_async_copy(
                w_ref.at[pl.ds(c * kc, kc), pl.ds(my * n_per, n_per)],
                wstage.at[c % 2], ws_sems.at[c % 2])
            cp.start()
            wcps[c] = cp

        start_wchunk(0)
        start_wchunk(1)

        def r_sub(h, j):
            o = lax.rem(my + (N_DEV - h), N_DEV)
            rows = pl.ds(o * m_per + j * qrt, qrt)
            return pltpu.make_async_remote_copy(
                src_ref=xg.at[rows, :], dst_ref=xg.at[rows, :],
                send_sem=rs_sems.at[2 * h + j], recv_sem=rr_sems.at[2 * h + j],
                device_id=(right,), device_id_type=pl.DeviceIdType.MESH)

        def l_sub(h, j):
            o = lax.rem(my + h, N_DEV)
            rows = pl.ds(o * m_per + half + j * qrt, qrt)
            return pltpu.make_async_remote_copy(
                src_ref=xg.at[rows, :], dst_ref=xg.at[rows, :],
                send_sem=ls_sems.at[2 * h + j], recv_sem=lr_sems.at[2 * h + j],
                device_id=(left,), device_id_type=pl.DeviceIdType.MESH)

        rsubs = {}
        lsubs = {}

        x_quarter_in(0)
        rsubs[(0, 0)] = r_sub(0, 0)
        rsubs[(0, 0)].start()
        x_quarter_in(2)
        lsubs[(0, 0)] = l_sub(0, 0)
        lsubs[(0, 0)].start()
        x_quarter_in(1)
        rsubs[(0, 1)] = r_sub(0, 1)
        rsubs[(0, 1)].start()
        x_quarter_in(3)
        lsubs[(0, 1)] = l_sub(0, 1)
        lsubs[(0, 1)].start()

        def w_chunk_in(c):
            wcps[c].wait()
            w8[pl.ds(c * kc, kc), :] = wstage[c % 2].astype(jnp.float8_e4m3fn)
            if c + 2 < W_CHUNKS:
                start_wchunk(c + 2)

        pending = [None, None]
        slot = [0]

        def compute_qrt(o, q):
            s = slot[0] & 1
            slot[0] += 1
            if pending[s] is not None:
                pending[s].wait()
            row = o * m_per + q * qrt
            acc = lax.dot_general(
                xg[pl.ds(row, qrt), :], w8[...],
                dimension_numbers=(((1,), (0,)), ((), ())),
                preferred_element_type=jnp.float32,
            )
            stage[s] = jnp.maximum(acc * scale, 0.0)
            cp = pltpu.make_async_copy(
                stage.at[s], out_ref.at[pl.ds(row, qrt), :],
                copy_sems.at[s])
            cp.start()
            pending[s] = cp

        def fwd_r(h, j):
            rsubs[(h - 1, j)].wait_recv()
            rsubs[(h, j)] = r_sub(h, j)
            rsubs[(h, j)].start()

        def fwd_l(h, j):
            lsubs[(h - 1, j)].wait_recv()
            lsubs[(h, j)] = l_sub(h, j)
            lsubs[(h, j)].start()

        o_am1 = lax.rem(my + (N_DEV - 1), N_DEV)
        o_ap1 = lax.rem(my + 1, N_DEV)
        o_2 = lax.rem(my + 2, N_DEV)

        for c in range(6):
            w_chunk_in(c)
        fwd_r(1, 0)
        fwd_l(1, 0)
        w_chunk_in(6)
        w_chunk_in(7)
        fwd_r(1, 1)
        fwd_l(1, 1)
        compute_qrt(my, 0)
        compute_qrt(my, 1)
        compute_qrt(my, 2)
        compute_qrt(my, 3)
        compute_qrt(o_am1, 0)
        fwd_r(2, 0)
        fwd_l(2, 0)
        compute_qrt(o_am1, 1)
        compute_qrt(o_ap1, 2)
        fwd_r(2, 1)
        fwd_l(2, 1)
        compute_qrt(o_ap1, 3)
        compute_qrt(o_2, 0)
        compute_qrt(o_2, 2)
        compute_qrt(o_2, 1)
        compute_qrt(o_2, 3)

        rsubs[(2, 0)].wait_recv()
        compute_qrt(o_ap1, 0)
        lsubs[(2, 0)].wait_recv()
        compute_qrt(o_am1, 2)
        rsubs[(2, 1)].wait_recv()
        compute_qrt(o_ap1, 1)
        lsubs[(2, 1)].wait_recv()
        compute_qrt(o_am1, 3)

        for h in range(N_HOPS):
            for j in range(2):
                rsubs[(h, j)].wait_send()
                lsubs[(h, j)].wait_send()

        for p in pending:
            if p is not None:
                p.wait()

    return pl.pallas_call(
        body,
        out_shape=jax.ShapeDtypeStruct((m_total, n_per), jnp.float32),
        in_specs=[
            pl.BlockSpec(memory_space=pl.ANY),
            pl.BlockSpec(memory_space=pl.ANY),
            pl.BlockSpec(memory_space=pltpu.SMEM),
            pl.BlockSpec(memory_space=pltpu.SMEM),
        ],
        out_specs=pl.BlockSpec(memory_space=pl.ANY),
        scratch_shapes=[
            pltpu.VMEM((m_total, k), jnp.float8_e4m3fn),
            pltpu.VMEM((k, n_per), jnp.float8_e4m3fn),
            pltpu.VMEM((2, kc, n_per), jnp.float32),
            pltpu.VMEM((4, qrt, k), jnp.float32),
            pltpu.VMEM((2, qrt, n_per), jnp.float32),
            pltpu.SemaphoreType.DMA((2,)),
            pltpu.SemaphoreType.DMA((4,)),
            pltpu.SemaphoreType.DMA((2 * N_HOPS,)),
            pltpu.SemaphoreType.DMA((2 * N_HOPS,)),
            pltpu.SemaphoreType.DMA((2 * N_HOPS,)),
            pltpu.SemaphoreType.DMA((2 * N_HOPS,)),
            pltpu.SemaphoreType.DMA((2,)),
        ],
        compiler_params=pltpu.CompilerParams(
            collective_id=0, vmem_limit_bytes=100 * 1024 * 1024),
    )(x, w_mat, scale_x, scale_w)
